# baseline (speedup 1.0000x reference)
import functools

import jax
import jax.numpy as jnp
from jax import lax
from jax.experimental import pallas as pl
from jax.experimental.pallas import tpu as pltpu
from jax.experimental.pallas import tpu_sc as plsc

SIZE = 2048
N_PAIRS = 65536
N_ROWS = 8192
DM = 2048
DMI = DM // 2
DS = 256
DSI = DS // 2
NW = 32
PPW = N_PAIRS // NW
CHUNK = 8
NBUF = 4
N_STEPS = PPW // (NBUF * CHUNK)

_mesh = plsc.VectorSubcoreMesh(core_axis_name="c", subcore_axis_name="s")


@functools.partial(
    pl.kernel,
    mesh=_mesh,
    compiler_params=pltpu.CompilerParams(needs_layout_passes=False),
    out_type=jax.ShapeDtypeStruct((N_PAIRS,), jnp.float32),
    scratch_types=(
        [pltpu.VMEM((CHUNK, DMI), jnp.int32),
         pltpu.VMEM((CHUNK, DMI), jnp.int32),
         pltpu.VMEM((CHUNK, DSI), jnp.int32),
         pltpu.VMEM((CHUNK, DSI), jnp.int32),
         ] * NBUF
        + [
        pltpu.VMEM((PPW,), jnp.int32),
        pltpu.VMEM((PPW,), jnp.int32),
        pltpu.VMEM((PPW,), jnp.int32),
        pltpu.VMEM((PPW,), jnp.int32),
        pltpu.VMEM((N_ROWS,), jnp.float32),
        pltpu.VMEM((N_ROWS,), jnp.float32),
        pltpu.VMEM((PPW,), jnp.float32),
        pltpu.VMEM((16, 16), jnp.float32),
        ]
        + [pltpu.SemaphoreType.DMA] * (4 * NBUF)
    ),
)
def _sc_logits(r_hbm, mw_hbm, rs_hbm, sw_hbm, am_hbm, bm_hbm, as_hbm, bs_hbm,
               mb_hbm, sb_hbm, out_hbm, *scratch):
    bufs = tuple(scratch[4 * i:4 * i + 4] + scratch[4 * NBUF + 8 + 4 * i:
                                                    4 * NBUF + 8 + 4 * i + 4]
                 for i in range(NBUF))
    (am_i, bm_i, as_i, bs_i, mb_v, sb_v, log_v, t_v
     ) = scratch[4 * NBUF:4 * NBUF + 8]
    wid = lax.axis_index("s") * 2 + lax.axis_index("c")
    base = wid * PPW
    pltpu.sync_copy(am_hbm.at[pl.ds(base, PPW)], am_i)
    pltpu.sync_copy(bm_hbm.at[pl.ds(base, PPW)], bm_i)
    pltpu.sync_copy(as_hbm.at[pl.ds(base, PPW)], as_i)
    pltpu.sync_copy(bs_hbm.at[pl.ds(base, PPW)], bs_i)
    pltpu.sync_copy(mb_hbm, mb_v)
    pltpu.sync_copy(sb_hbm, sb_v)

    def copies(buf, off):
        a_v, b_v, sa_v, sb_v, se_a, se_b, se_sa, se_sb = buf
        return (
            pltpu.make_async_copy(r_hbm.at[am_i.at[pl.ds(off, CHUNK)]], a_v, se_a),
            pltpu.make_async_copy(mw_hbm.at[bm_i.at[pl.ds(off, CHUNK)]], b_v, se_b),
            pltpu.make_async_copy(rs_hbm.at[as_i.at[pl.ds(off, CHUNK)]], sa_v, se_sa),
            pltpu.make_async_copy(sw_hbm.at[bs_i.at[pl.ds(off, CHUNK)]], sb_v, se_sb),
        )

    def issue(buf, off):
        for c in copies(buf, off):
            c.start()

    def drain(buf, off):
        for c in copies(buf, off):
            c.wait()

    def compute8(buf, trow):
        a_v, b_v, sa_v, sb_v = buf[:4]
        z = jnp.zeros((16,), jnp.float32)
        for p in range(CHUNK):
            def dot_main(k, accs):
                c0, c1, c2, c3 = accs
                o = k * 32
                a32 = plsc.bitcast(a_v[p, pl.ds(o, 16)], jnp.bfloat16)
                b32 = plsc.bitcast(b_v[p, pl.ds(o, 16)], jnp.bfloat16)
                al, ah = plsc.unpack(a32, format=plsc.PackFormat.INTERLEAVED,
                                     preferred_element_type=jnp.float32)
                bl, bh = plsc.unpack(b32, format=plsc.PackFormat.INTERLEAVED,
                                     preferred_element_type=jnp.float32)
                c0 += al * bl
                c1 += ah * bh
                a32 = plsc.bitcast(a_v[p, pl.ds(o + 16, 16)], jnp.bfloat16)
                b32 = plsc.bitcast(b_v[p, pl.ds(o + 16, 16)], jnp.bfloat16)
                al, ah = plsc.unpack(a32, format=plsc.PackFormat.INTERLEAVED,
                                     preferred_element_type=jnp.float32)
                bl, bh = plsc.unpack(b32, format=plsc.PackFormat.INTERLEAVED,
                                     preferred_element_type=jnp.float32)
                c2 += al * bl
                c3 += ah * bh
                return (c0, c1, c2, c3)

            acc = lax.fori_loop(0, DMI // 32, dot_main, (z, z, z, z))
            c0, c1, c2, c3 = acc
            for k in range(DSI // 32):
                o = k * 32
                a32 = plsc.bitcast(sa_v[p, pl.ds(o, 16)], jnp.bfloat16)
                b32 = plsc.bitcast(sb_v[p, pl.ds(o, 16)], jnp.bfloat16)
                al, ah = plsc.unpack(a32, format=plsc.PackFormat.INTERLEAVED,
                                     preferred_element_type=jnp.float32)
                bl, bh = plsc.unpack(b32, format=plsc.PackFormat.INTERLEAVED,
                                     preferred_element_type=jnp.float32)
                c0 += al * bl
                c1 += ah * bh
                a32 = plsc.bitcast(sa_v[p, pl.ds(o + 16, 16)], jnp.bfloat16)
                b32 = plsc.bitcast(sb_v[p, pl.ds(o + 16, 16)], jnp.bfloat16)
                al, ah = plsc.unpack(a32, format=plsc.PackFormat.INTERLEAVED,
                                     preferred_element_type=jnp.float32)
                bl, bh = plsc.unpack(b32, format=plsc.PackFormat.INTERLEAVED,
                                     preferred_element_type=jnp.float32)
                c2 += al * bl
                c3 += ah * bh
            t_v[trow + p, :] = (c0 + c1) + (c2 + c3)

    for b in range(NBUF):
        issue(bufs[b], b * CHUNK)

    lane = lax.iota(jnp.int32, 16)

    def finalize(off):
        def col_sum(i, vec):
            return vec + plsc.load_gather(t_v, [lane, jnp.full((16,), i, jnp.int32)])

        out_vec = lax.fori_loop(0, 16, col_sum, jnp.zeros((16,), jnp.float32))
        bias = (plsc.load_gather(mb_v, [bm_i[pl.ds(off, 16)]])
                + plsc.load_gather(sb_v, [bs_i[pl.ds(off, 16)]]))
        log_v[pl.ds(off, 16)] = out_vec + bias

    def step(g, carry):
        off0 = g * NBUF * CHUNK
        for b in range(NBUF):
            ob = off0 + b * CHUNK
            drain(bufs[b], ob)
            compute8(bufs[b], (b % 2) * CHUNK)

            @pl.when(g < N_STEPS - 1)
            def _(b=b, ob=ob):
                issue(bufs[b], ob + NBUF * CHUNK)

            if b % 2 == 1:
                finalize(off0 + (b // 2) * 2 * CHUNK)
        return carry

    lax.fori_loop(0, N_STEPS, step, 0)
    pltpu.sync_copy(log_v, out_hbm.at[pl.ds(base, PPW)])


def _rne(x_u32):
    return (x_u32 + jnp.uint32(0x7FFF) + ((x_u32 >> 16) & jnp.uint32(1))) >> 16


def _pack_body(mw_ref, r_ref, sw_ref, mwi_ref, rbi_ref, rsi_ref, swi_ref):
    def pack(lo, hi):
        return (lo | (hi << 16)).astype(jnp.int32)

    w = _rne(lax.bitcast_convert_type(mw_ref[...], jnp.uint32)[:, 1:2049])
    mwi_ref[...] = pack(w[:, :1024], w[:, 1024:])
    rbits = _rne(lax.bitcast_convert_type(r_ref[...], jnp.uint32))
    rbi_ref[...] = pack(rbits[:, :1024], rbits[:, 1024:])
    col = lax.broadcasted_iota(jnp.uint32, (512, DSI), 1)
    rs_hi = jnp.where(col < 72, rbits[:, 128:128 + DSI], jnp.uint32(0))
    rsi_ref[...] = pack(rbits[:, :DSI], rs_hi)
    swb = _rne(lax.bitcast_convert_type(sw_ref[...], jnp.uint32)[:, 1:201])
    sw_hi = jnp.where(col < 72,
                      jnp.concatenate(
                          [swb[:, 128:200],
                           jnp.zeros((512, DSI - 72), jnp.uint32)], axis=1),
                      jnp.uint32(0))
    swi_ref[...] = pack(swb[:, :DSI], sw_hi)


_tc_pack = pl.pallas_call(
    _pack_body,
    grid=(16,),
    in_specs=[
        pl.BlockSpec((512, 2049), lambda i: (i, 0)),
        pl.BlockSpec((512, 2048), lambda i: (i, 0)),
        pl.BlockSpec((512, 201), lambda i: (i, 0)),
    ],
    out_specs=[
        pl.BlockSpec((512, DMI), lambda i: (i, 0)),
        pl.BlockSpec((512, DMI), lambda i: (i, 0)),
        pl.BlockSpec((512, DSI), lambda i: (i, 0)),
        pl.BlockSpec((512, DSI), lambda i: (i, 0)),
    ],
    out_shape=[
        jax.ShapeDtypeStruct((N_ROWS, DMI), jnp.int32),
        jax.ShapeDtypeStruct((N_ROWS, DMI), jnp.int32),
        jax.ShapeDtypeStruct((N_ROWS, DSI), jnp.int32),
        jax.ShapeDtypeStruct((N_ROWS, DSI), jnp.int32),
    ],
)


def _loss_body(lg_ref, lab_ref, fr_ref, out_ref):
    x1 = lg_ref[:256, :]
    y1 = lab_ref[:256, :]
    p = jax.nn.sigmoid(x1) * fr_ref[...]
    ln_p = jnp.maximum(jnp.log(p), -100.0)
    ln_1mp = jnp.maximum(jnp.log(1.0 - p), -100.0)
    frac_loss = jnp.sum(-(y1 * ln_p + (1.0 - y1) * ln_1mp))
    x2 = lg_ref[256:, :]
    y2 = lab_ref[256:, :]
    nonfrac_loss = jnp.sum(jnp.maximum(x2, 0.0) - x2 * y2
                           + jnp.log1p(jnp.exp(-jnp.abs(x2))))
    out_ref[...] = jnp.reshape((frac_loss + nonfrac_loss) / N_PAIRS, (1, 1))


_tc_loss = pl.pallas_call(
    _loss_body,
    out_shape=jax.ShapeDtypeStruct((1, 1), jnp.float32),
)


@jax.jit
def kernel(rnn_output, labels, fracs, main_w, sub_w, indices, other_indices):
    r = rnn_output.reshape(-1, SIZE)
    mw, rbi, rs, sw = _tc_pack(main_w, r, sub_w)
    mb = main_w[:N_ROWS, 0]
    sb = sub_w[:N_ROWS, 0]
    am = other_indices[:, 0].astype(jnp.int32)
    bm = other_indices[:, 1].astype(jnp.int32)
    a_s = indices[:, 0].astype(jnp.int32)
    b_s = indices[:, 1].astype(jnp.int32)
    logits = _sc_logits(rbi, mw, rs, sw, am, bm, a_s, b_s, mb, sb)
    loss = _tc_loss(logits.reshape(512, 128), labels.reshape(512, 128),
                    fracs.reshape(256, 128))
    return logits, loss[0, 0]


# speedup vs baseline: 1.5483x; 1.5483x over previous
import functools

import jax
import jax.numpy as jnp
from jax import lax
from jax.experimental import pallas as pl
from jax.experimental.pallas import tpu as pltpu
from jax.experimental.pallas import tpu_sc as plsc

SIZE = 2048
N_PAIRS = 65536
N_ROWS = 8192
DM = 2048
DMI = DM // 2
DS = 256
NW = 32
PPW = N_PAIRS // NW
CHUNK = 8
N_STEPS = PPW // (2 * CHUNK)

_mesh = plsc.VectorSubcoreMesh(core_axis_name="c", subcore_axis_name="s")


@functools.partial(
    pl.kernel,
    mesh=_mesh,
    compiler_params=pltpu.CompilerParams(needs_layout_passes=False),
    out_type=jax.ShapeDtypeStruct((N_PAIRS,), jnp.float32),
    scratch_types=[
        pltpu.VMEM((CHUNK, DMI), jnp.int32),
        pltpu.VMEM((CHUNK, DMI), jnp.int32),
        pltpu.VMEM((CHUNK, DS), jnp.float32),
        pltpu.VMEM((CHUNK, DS), jnp.float32),
        pltpu.VMEM((CHUNK, DMI), jnp.int32),
        pltpu.VMEM((CHUNK, DMI), jnp.int32),
        pltpu.VMEM((CHUNK, DS), jnp.float32),
        pltpu.VMEM((CHUNK, DS), jnp.float32),
        pltpu.VMEM((PPW,), jnp.int32),
        pltpu.VMEM((PPW,), jnp.int32),
        pltpu.VMEM((PPW,), jnp.int32),
        pltpu.VMEM((PPW,), jnp.int32),
        pltpu.VMEM((N_ROWS,), jnp.float32),
        pltpu.VMEM((N_ROWS,), jnp.float32),
        pltpu.VMEM((PPW,), jnp.float32),
        pltpu.VMEM((16, 16), jnp.float32),
        pltpu.SemaphoreType.DMA,
        pltpu.SemaphoreType.DMA,
        pltpu.SemaphoreType.DMA,
        pltpu.SemaphoreType.DMA,
        pltpu.SemaphoreType.DMA,
        pltpu.SemaphoreType.DMA,
        pltpu.SemaphoreType.DMA,
        pltpu.SemaphoreType.DMA,
    ],
)
def _sc_logits(r_hbm, mw_hbm, rs_hbm, sw_hbm, am_hbm, bm_hbm, as_hbm, bs_hbm,
               mb_hbm, sb_hbm, out_hbm,
               a0_v, b0_v, s0a_v, s0b_v, a1_v, b1_v, s1a_v, s1b_v,
               am_i, bm_i, as_i, bs_i, mb_v, sb_v,
               log_v, t_v,
               sem_a0, sem_b0, sem_sa0, sem_sb0,
               sem_a1, sem_b1, sem_sa1, sem_sb1):
    wid = lax.axis_index("s") * 2 + lax.axis_index("c")
    base = wid * PPW
    pltpu.sync_copy(am_hbm.at[pl.ds(base, PPW)], am_i)
    pltpu.sync_copy(bm_hbm.at[pl.ds(base, PPW)], bm_i)
    pltpu.sync_copy(as_hbm.at[pl.ds(base, PPW)], as_i)
    pltpu.sync_copy(bs_hbm.at[pl.ds(base, PPW)], bs_i)
    pltpu.sync_copy(mb_hbm, mb_v)
    pltpu.sync_copy(sb_hbm, sb_v)

    bufs = ((a0_v, b0_v, s0a_v, s0b_v, sem_a0, sem_b0, sem_sa0, sem_sb0),
            (a1_v, b1_v, s1a_v, s1b_v, sem_a1, sem_b1, sem_sa1, sem_sb1))

    def copies(buf, off):
        a_v, b_v, sa_v, sb_v, se_a, se_b, se_sa, se_sb = buf
        return (
            pltpu.make_async_copy(r_hbm.at[am_i.at[pl.ds(off, CHUNK)]], a_v, se_a),
            pltpu.make_async_copy(mw_hbm.at[bm_i.at[pl.ds(off, CHUNK)]], b_v, se_b),
            pltpu.make_async_copy(rs_hbm.at[as_i.at[pl.ds(off, CHUNK)]], sa_v, se_sa),
            pltpu.make_async_copy(sw_hbm.at[bs_i.at[pl.ds(off, CHUNK)]], sb_v, se_sb),
        )

    def issue(buf, off):
        for c in copies(buf, off):
            c.start()

    def drain(buf, off):
        for c in copies(buf, off):
            c.wait()

    def compute8(buf, trow):
        a_v, b_v, sa_v, sb_v = buf[:4]
        z = jnp.zeros((16,), jnp.float32)
        for p in range(CHUNK):
            def dot_main(k, accs):
                c0, c1, c2, c3 = accs
                o = k * 32
                a32 = plsc.bitcast(a_v[p, pl.ds(o, 16)], jnp.bfloat16)
                b32 = plsc.bitcast(b_v[p, pl.ds(o, 16)], jnp.bfloat16)
                al, ah = plsc.unpack(a32, format=plsc.PackFormat.INTERLEAVED,
                                     preferred_element_type=jnp.float32)
                bl, bh = plsc.unpack(b32, format=plsc.PackFormat.INTERLEAVED,
                                     preferred_element_type=jnp.float32)
                c0 += al * bl
                c1 += ah * bh
                a32 = plsc.bitcast(a_v[p, pl.ds(o + 16, 16)], jnp.bfloat16)
                b32 = plsc.bitcast(b_v[p, pl.ds(o + 16, 16)], jnp.bfloat16)
                al, ah = plsc.unpack(a32, format=plsc.PackFormat.INTERLEAVED,
                                     preferred_element_type=jnp.float32)
                bl, bh = plsc.unpack(b32, format=plsc.PackFormat.INTERLEAVED,
                                     preferred_element_type=jnp.float32)
                c2 += al * bl
                c3 += ah * bh
                return (c0, c1, c2, c3)

            acc = lax.fori_loop(0, DMI // 32, dot_main, (z, z, z, z))
            c0, c1, c2, c3 = acc
            for k in range(DS // 64):
                o = k * 64
                c0 += sa_v[p, pl.ds(o, 16)] * sb_v[p, pl.ds(o, 16)]
                c1 += sa_v[p, pl.ds(o + 16, 16)] * sb_v[p, pl.ds(o + 16, 16)]
                c2 += sa_v[p, pl.ds(o + 32, 16)] * sb_v[p, pl.ds(o + 32, 16)]
                c3 += sa_v[p, pl.ds(o + 48, 16)] * sb_v[p, pl.ds(o + 48, 16)]
            t_v[trow + p, :] = (c0 + c1) + (c2 + c3)

    issue(bufs[0], 0)
    issue(bufs[1], CHUNK)

    lane = lax.iota(jnp.int32, 16)

    def step(g, carry):
        off0 = g * 2 * CHUNK
        drain(bufs[0], off0)
        compute8(bufs[0], 0)

        @pl.when(g < N_STEPS - 1)
        def _():
            issue(bufs[0], off0 + 2 * CHUNK)

        drain(bufs[1], off0 + CHUNK)
        compute8(bufs[1], CHUNK)

        @pl.when(g < N_STEPS - 1)
        def _():
            issue(bufs[1], off0 + 3 * CHUNK)

        def col_sum(i, vec):
            return vec + plsc.load_gather(t_v, [lane, jnp.full((16,), i, jnp.int32)])

        out_vec = lax.fori_loop(0, 16, col_sum, jnp.zeros((16,), jnp.float32))
        bias = (plsc.load_gather(mb_v, [bm_i[pl.ds(off0, 16)]])
                + plsc.load_gather(sb_v, [bs_i[pl.ds(off0, 16)]]))
        log_v[pl.ds(off0, 16)] = out_vec + bias
        return carry

    lax.fori_loop(0, N_STEPS, step, 0)
    pltpu.sync_copy(log_v, out_hbm.at[pl.ds(base, PPW)])


def _pack_body(mw_ref, r_ref, mwi_ref, rbi_ref):
    def pack(x_u32):
        b = (x_u32 + jnp.uint32(0x7FFF) + ((x_u32 >> 16) & jnp.uint32(1))) >> 16
        lo = b[:, :1024]
        hi = b[:, 1024:]
        return (lo | (hi << 16)).astype(jnp.int32)

    w = lax.bitcast_convert_type(mw_ref[...], jnp.uint32)
    mwi_ref[...] = pack(w[:, 1:2049])
    rbits = lax.bitcast_convert_type(r_ref[...], jnp.uint32)
    rbi_ref[...] = pack(rbits)


_tc_pack = pl.pallas_call(
    _pack_body,
    grid=(16,),
    in_specs=[
        pl.BlockSpec((512, 2049), lambda i: (i, 0)),
        pl.BlockSpec((512, 2048), lambda i: (i, 0)),
    ],
    out_specs=[
        pl.BlockSpec((512, 1024), lambda i: (i, 0)),
        pl.BlockSpec((512, 1024), lambda i: (i, 0)),
    ],
    out_shape=[
        jax.ShapeDtypeStruct((N_ROWS, DMI), jnp.int32),
        jax.ShapeDtypeStruct((N_ROWS, DMI), jnp.int32),
    ],
)


def _loss_body(lg_ref, lab_ref, fr_ref, out_ref):
    x1 = lg_ref[:256, :]
    y1 = lab_ref[:256, :]
    p = jax.nn.sigmoid(x1) * fr_ref[...]
    ln_p = jnp.maximum(jnp.log(p), -100.0)
    ln_1mp = jnp.maximum(jnp.log(1.0 - p), -100.0)
    frac_loss = jnp.sum(-(y1 * ln_p + (1.0 - y1) * ln_1mp))
    x2 = lg_ref[256:, :]
    y2 = lab_ref[256:, :]
    nonfrac_loss = jnp.sum(jnp.maximum(x2, 0.0) - x2 * y2
                           + jnp.log1p(jnp.exp(-jnp.abs(x2))))
    out_ref[...] = jnp.reshape((frac_loss + nonfrac_loss) / N_PAIRS, (1, 1))


_tc_loss = pl.pallas_call(
    _loss_body,
    out_shape=jax.ShapeDtypeStruct((1, 1), jnp.float32),
)


@jax.jit
def kernel(rnn_output, labels, fracs, main_w, sub_w, indices, other_indices):
    r = rnn_output.reshape(-1, SIZE)
    mw, rbi = _tc_pack(main_w, r)
    mb = main_w[:N_ROWS, 0]
    zpad = jnp.zeros((N_ROWS, DS - 200), jnp.float32)
    rs = jnp.concatenate([r[:, :200], zpad], axis=1)
    sw = jnp.concatenate([sub_w[:N_ROWS, 1:201], zpad], axis=1)
    sb = sub_w[:N_ROWS, 0]
    am = other_indices[:, 0].astype(jnp.int32)
    bm = other_indices[:, 1].astype(jnp.int32)
    a_s = indices[:, 0].astype(jnp.int32)
    b_s = indices[:, 1].astype(jnp.int32)
    logits = _sc_logits(rbi, mw, rs, sw, am, bm, a_s, b_s, mb, sb)
    loss = _tc_loss(logits.reshape(512, 128), labels.reshape(512, 128),
                    fracs.reshape(256, 128))
    return logits, loss[0, 0]
